# baseline (speedup 1.0000x reference)
import numpy as np
import jax
import jax.numpy as jnp
from jax.experimental import pallas as pl
from jax.experimental.pallas import tpu as pltpu


def _make_prep_body(C, base, W):
    def _body(c_ref, w_ref, b_ref, rxc_ref, cw_ref, cb_ref,
              h_ref, m_ref, bias_ref):
        y = jnp.dot(c_ref[...], w_ref[...], preferred_element_type=jnp.float32)
        y = y + b_ref[...]
        h_ref[...] = y * (1.0 / (1.0 + jnp.exp(-y)))
        rxc = rxc_ref[...]
        for ky in range(3):
            for c in range(C):
                acc = None
                for kx in range(3):
                    w_sc = (cw_ref[c * 36 + 0 + ky * 3 + kx]
                            + cw_ref[c * 36 + 9 + ky * 3 + kx]
                            + cw_ref[c * 36 + 18 + ky * 3 + kx]
                            + cw_ref[c * 36 + 27 + ky * 3 + kx])
                    term = w_sc * rxc[:, kx * W:(kx + 1) * W]
                    acc = term if acc is None else acc + term
                m_ref[ky * base:(ky + 1) * base, c * W:(c + 1) * W] = (
                    acc.astype(jnp.bfloat16))
        for c in range(C):
            bias_ref[:, c * W:(c + 1) * W] = jnp.full((1, W), cb_ref[c],
                                                      jnp.float32)
    return _body


def _prep(c, w, b, rxc, conv_w, conv_b):
    B = c.shape[0]
    D = w.shape[1]
    base = rxc.shape[0]
    W = rxc.shape[1] // 3
    C = conv_b.shape[0]
    K3 = 3 * base
    vmem = pl.BlockSpec(memory_space=pltpu.MemorySpace.VMEM)
    smem = pl.BlockSpec(memory_space=pltpu.MemorySpace.SMEM)
    return pl.pallas_call(
        _make_prep_body(C, base, W),
        out_shape=(jax.ShapeDtypeStruct((B, D), jnp.float32),
                   jax.ShapeDtypeStruct((K3, C * W), jnp.bfloat16),
                   jax.ShapeDtypeStruct((1, C * W), jnp.float32)),
        in_specs=[vmem, vmem, vmem, vmem, smem, smem],
        out_specs=[vmem, vmem, vmem],
    )(c, w, b.reshape(1, D), rxc, conv_w.reshape(-1), conv_b)


def _bilinear_matrix(out_size, in_size):
    scale = in_size / out_size
    M = np.zeros((out_size, in_size), dtype=np.float32)
    for i in range(out_size):
        src = (i + 0.5) * scale - 0.5
        src = max(src, 0.0)
        x0 = min(int(np.floor(src)), in_size - 1)
        x1 = min(x0 + 1, in_size - 1)
        l1 = src - x0
        M[i, x0] += 1.0 - l1
        M[i, x1] += l1
    return M


def _upsample_pad_matrix(R, base):
    s = R // base
    M = np.zeros((R + 2, base), dtype=np.float32)
    for i in range(R):
        M[i + 1, i // s] = 1.0
    return M


def _fold_factors(base, R, H, W):
    up = _upsample_pad_matrix(R, base)
    wy = _bilinear_matrix(H, R)
    wx = _bilinear_matrix(W, R)
    ly = np.stack([wy @ up[k:k + R, :] for k in range(3)], axis=0)
    rx = np.stack([(wx @ up[k:k + R, :]).T for k in range(3)], axis=0)
    return ly, rx


def _make_fused_body(C, base, W, BB):
    K3 = 3 * base

    def _body(h_ref, l_ref, m_ref, bias_ref, x_ref, xo_ref, co_ref):
        z = jnp.zeros((base, base), jnp.bfloat16)
        for j in range(BB):
            hb = h_ref[j].astype(jnp.bfloat16)
            hd = jnp.concatenate([
                jnp.concatenate([hb, z, z], axis=1),
                jnp.concatenate([z, hb, z], axis=1),
                jnp.concatenate([z, z, hb], axis=1),
            ], axis=0)
            t = jnp.dot(l_ref[...], hd,
                        preferred_element_type=jnp.float32)
            acc = jnp.dot(t.astype(jnp.bfloat16), m_ref[...],
                          preferred_element_type=jnp.float32)
            acc = acc + bias_ref[...]
            for c in range(C):
                ci = acc[:, c * W:(c + 1) * W]
                co_ref[j, c] = ci
                xo_ref[j, c] = x_ref[j, c] + ci
    return _body


def kernel(x, c, lin_w, lin_b, conv_w, conv_b):
    B, C, H, W = x.shape
    base = 64
    R = 256
    K3 = 3 * base

    ly, rx = _fold_factors(base, R, H, W)
    l_cat = jnp.asarray(
        np.concatenate([ly[0], ly[1], ly[2]], axis=1), dtype=jnp.bfloat16
    )
    rxc = jnp.asarray(
        np.concatenate([rx[0], rx[1], rx[2]], axis=1))

    h, m_all, bias = _prep(c, lin_w, lin_b, rxc, conv_w, conv_b)
    h = h.reshape(B, base, base)

    BB = 8 if B % 8 == 0 else 1
    out_shapes = (jax.ShapeDtypeStruct((B, C, H, W), jnp.float32),
                  jax.ShapeDtypeStruct((B, C, H, W), jnp.float32))
    xo, co = pl.pallas_call(
        _make_fused_body(C, base, W, BB),
        out_shape=out_shapes,
        grid=(B // BB,),
        in_specs=[
            pl.BlockSpec((BB, base, base), lambda b: (b, 0, 0)),
            pl.BlockSpec((H, K3), lambda b: (0, 0)),
            pl.BlockSpec((K3, C * W), lambda b: (0, 0)),
            pl.BlockSpec((1, C * W), lambda b: (0, 0)),
            pl.BlockSpec((BB, C, H, W), lambda b: (b, 0, 0, 0)),
        ],
        out_specs=[
            pl.BlockSpec((BB, C, H, W), lambda b: (b, 0, 0, 0)),
            pl.BlockSpec((BB, C, H, W), lambda b: (b, 0, 0, 0)),
        ],
        compiler_params=pltpu.CompilerParams(
            dimension_semantics=("parallel",),
            vmem_limit_bytes=60 * 1024 * 1024),
    )(h, l_cat, m_all, bias, x)
    return xo, co


# speedup vs baseline: 1.1033x; 1.1033x over previous
import numpy as np
import jax
import jax.numpy as jnp
from jax.experimental import pallas as pl
from jax.experimental.pallas import tpu as pltpu


def _bilinear_matrix(out_size, in_size):
    scale = in_size / out_size
    M = np.zeros((out_size, in_size), dtype=np.float32)
    for i in range(out_size):
        src = (i + 0.5) * scale - 0.5
        src = max(src, 0.0)
        x0 = min(int(np.floor(src)), in_size - 1)
        x1 = min(x0 + 1, in_size - 1)
        l1 = src - x0
        M[i, x0] += 1.0 - l1
        M[i, x1] += l1
    return M


def _upsample_pad_matrix(R, base):
    s = R // base
    M = np.zeros((R + 2, base), dtype=np.float32)
    for i in range(R):
        M[i + 1, i // s] = 1.0
    return M


def _fold_factors(base, R, H, W):
    up = _upsample_pad_matrix(R, base)
    wy = _bilinear_matrix(H, R)
    wx = _bilinear_matrix(W, R)
    ly = np.stack([wy @ up[k:k + R, :] for k in range(3)], axis=0)
    rx = np.stack([(wx @ up[k:k + R, :]).T for k in range(3)], axis=0)
    return ly, rx


def _make_body(C, base, W, BB):
    K3 = 3 * base

    def _body(c_ref, w_ref, b_ref, rxc_ref, l_ref, cw_ref, cb_ref,
              x_ref, xo_ref, co_ref,
              h_scr, m_scr, bias_scr):
        step = pl.program_id(0)

        @pl.when(step == 0)
        def _prep():
            y = jnp.dot(c_ref[...], w_ref[...],
                        preferred_element_type=jnp.float32)
            y = y + b_ref[...]
            y = y * (1.0 / (1.0 + jnp.exp(-y)))
            for i in range(base):
                h_scr[:, pl.ds(i, 1), :] = y[:, base * i:base * (i + 1)][:, None, :]
            rxc = rxc_ref[...]
            for ky in range(3):
                for ch in range(C):
                    acc = None
                    for kx in range(3):
                        w_sc = (cw_ref[ch * 36 + 0 + ky * 3 + kx]
                                + cw_ref[ch * 36 + 9 + ky * 3 + kx]
                                + cw_ref[ch * 36 + 18 + ky * 3 + kx]
                                + cw_ref[ch * 36 + 27 + ky * 3 + kx])
                        term = w_sc * rxc[:, kx * W:(kx + 1) * W]
                        acc = term if acc is None else acc + term
                    m_scr[ky * base:(ky + 1) * base, ch * W:(ch + 1) * W] = (
                        acc.astype(jnp.bfloat16))
            for ch in range(C):
                bias_scr[:, ch * W:(ch + 1) * W] = jnp.full(
                    (1, W), cb_ref[ch], jnp.float32)

        m_all = m_scr[...]
        bias = bias_scr[...]
        z = jnp.zeros((base, base), jnp.bfloat16)
        for j in range(BB):
            hb = h_scr[step * BB + j].astype(jnp.bfloat16)
            hd = jnp.concatenate([
                jnp.concatenate([hb, z, z], axis=1),
                jnp.concatenate([z, hb, z], axis=1),
                jnp.concatenate([z, z, hb], axis=1),
            ], axis=0)
            t = jnp.dot(l_ref[...], hd,
                        preferred_element_type=jnp.float32)
            acc = jnp.dot(t.astype(jnp.bfloat16), m_all,
                          preferred_element_type=jnp.float32)
            acc = acc + bias
            for ch in range(C):
                ci = acc[:, ch * W:(ch + 1) * W]
                co_ref[j, ch] = ci
                xo_ref[j, ch] = x_ref[j, ch] + ci
    return _body


def kernel(x, c, lin_w, lin_b, conv_w, conv_b):
    B, C, H, W = x.shape
    base = 64
    R = 256
    K3 = 3 * base
    D = lin_w.shape[1]

    ly, rx = _fold_factors(base, R, H, W)
    l_cat = jnp.asarray(
        np.concatenate([ly[0], ly[1], ly[2]], axis=1), dtype=jnp.bfloat16
    )
    rxc = jnp.asarray(
        np.concatenate([rx[0], rx[1], rx[2]], axis=1))

    BB = 8 if B % 8 == 0 else 1
    cmap = lambda b: (0, 0)
    out_shapes = (jax.ShapeDtypeStruct((B, C, H, W), jnp.float32),
                  jax.ShapeDtypeStruct((B, C, H, W), jnp.float32))
    xo, co = pl.pallas_call(
        _make_body(C, base, W, BB),
        out_shape=out_shapes,
        grid=(B // BB,),
        in_specs=[
            pl.BlockSpec((B, c.shape[1]), cmap),
            pl.BlockSpec((c.shape[1], D), cmap),
            pl.BlockSpec((1, D), cmap),
            pl.BlockSpec((base, 3 * W), cmap),
            pl.BlockSpec((H, K3), cmap),
            pl.BlockSpec(memory_space=pltpu.MemorySpace.SMEM),
            pl.BlockSpec(memory_space=pltpu.MemorySpace.SMEM),
            pl.BlockSpec((BB, C, H, W), lambda b: (b, 0, 0, 0)),
        ],
        out_specs=[
            pl.BlockSpec((BB, C, H, W), lambda b: (b, 0, 0, 0)),
            pl.BlockSpec((BB, C, H, W), lambda b: (b, 0, 0, 0)),
        ],
        scratch_shapes=[
            pltpu.VMEM((B, base, base), jnp.float32),
            pltpu.VMEM((K3, C * W), jnp.bfloat16),
            pltpu.VMEM((1, C * W), jnp.float32),
        ],
        compiler_params=pltpu.CompilerParams(
            dimension_semantics=("arbitrary",),
            vmem_limit_bytes=60 * 1024 * 1024),
    )(c, lin_w, lin_b.reshape(1, D), rxc, l_cat,
      conv_w.reshape(-1), conv_b, x)
    return xo, co
